# baseline (device time: 27984 ns/iter reference)
import jax
import jax.numpy as jnp
from jax import lax
from jax.experimental import pallas as pl
from jax.experimental.pallas import tpu as pltpu

N_DEV = 16


def kernel(x, w_mat):
    m_per, k = x.shape
    _, n_per = w_mat.shape

    def body(x_ref, w_ref, out_ref, xfull_ref, src_ref,
             send_sems, recv_sems, copy_sem):
        my = lax.axis_index("i")

        src_ref[...] = x_ref[...].astype(jnp.bfloat16)

        barrier = pltpu.get_barrier_semaphore()
        for j in range(1, N_DEV):
            pl.semaphore_signal(
                barrier, inc=1,
                device_id=((my + j) % N_DEV,),
                device_id_type=pl.DeviceIdType.MESH,
            )
        pl.semaphore_wait(barrier, N_DEV - 1)

        sends = []
        for j in range(1, N_DEV):
            rdma = pltpu.make_async_remote_copy(
                src_ref=src_ref,
                dst_ref=xfull_ref.at[my],
                send_sem=send_sems.at[j - 1],
                recv_sem=recv_sems.at[my],
                device_id=((my + j) % N_DEV,),
                device_id_type=pl.DeviceIdType.MESH,
            )
            rdma.start()
            sends.append(rdma)

        own = pltpu.make_async_copy(src_ref, xfull_ref.at[my], copy_sem)
        own.start()
        own.wait()

        for j in range(1, N_DEV):
            o = (my + j) % N_DEV
            recv = pltpu.make_async_remote_copy(
                src_ref=src_ref,
                dst_ref=xfull_ref.at[o],
                send_sem=send_sems.at[j - 1],
                recv_sem=recv_sems.at[o],
                device_id=(my,),
                device_id_type=pl.DeviceIdType.MESH,
            )
            recv.wait_recv()

        xf = xfull_ref[...].reshape(N_DEV * m_per, k)
        wb = w_ref[...].astype(jnp.bfloat16)
        out_ref[...] = jnp.dot(xf, wb, preferred_element_type=jnp.float32)

        for rdma in sends:
            rdma.wait_send()

    return pl.pallas_call(
        body,
        out_shape=jax.ShapeDtypeStruct((N_DEV * m_per, n_per), jnp.float32),
        in_specs=[
            pl.BlockSpec(memory_space=pltpu.VMEM),
            pl.BlockSpec(memory_space=pltpu.VMEM),
        ],
        out_specs=pl.BlockSpec(memory_space=pltpu.VMEM),
        scratch_shapes=[
            pltpu.VMEM((N_DEV, m_per, k), jnp.bfloat16),
            pltpu.VMEM((m_per, k), jnp.bfloat16),
            pltpu.SemaphoreType.DMA((N_DEV - 1,)),
            pltpu.SemaphoreType.DMA((N_DEV,)),
            pltpu.SemaphoreType.DMA,
        ],
        compiler_params=pltpu.CompilerParams(collective_id=0),
    )(x, w_mat)


# device time: 21949 ns/iter; 1.2750x vs baseline; 1.2750x over previous
import jax
import jax.numpy as jnp
from jax import lax
from jax.experimental import pallas as pl
from jax.experimental.pallas import tpu as pltpu

N_DEV = 16
NZ = 4
NQ = 4
N_SEND = 27


def kernel(x, w_mat):
    m_per, k = x.shape
    _, n_per = w_mat.shape

    def body(x_ref, w_ref, out_ref, xfull_ref, src_ref,
             send_sems, up_sems, dn_sems, ipx_sems, ipy_sems, ipd_sems,
             copy_sem):
        my = lax.axis_index("i")
        z = my // NQ
        q = my % NQ
        x_nbr = NQ * z + (q ^ 1)
        y_nbr = NQ * z + (3 - q)
        d_nbr = NQ * z + (q ^ 2)
        up = jnp.minimum(my + NQ, N_DEV - 1)
        dn = jnp.maximum(my - NQ, 0)
        has_up = z < NZ - 1
        has_dn = z > 0

        src_ref[...] = x_ref[...].astype(jnp.bfloat16)

        barrier = pltpu.get_barrier_semaphore()
        for tgt in (x_nbr, y_nbr, d_nbr):
            pl.semaphore_signal(barrier, inc=1, device_id=(tgt,),
                                device_id_type=pl.DeviceIdType.MESH)
        for cond, tgt in ((has_up, up), (has_dn, dn)):
            @pl.when(cond)
            def _(tgt=tgt):
                pl.semaphore_signal(barrier, inc=1, device_id=(tgt,),
                                    device_id_type=pl.DeviceIdType.MESH)

            @pl.when(jnp.logical_not(cond))
            def _():
                pl.semaphore_signal(barrier, inc=1, device_id=(my,),
                                    device_id_type=pl.DeviceIdType.MESH)
        pl.semaphore_wait(barrier, 5)

        own = pltpu.make_async_copy(src_ref, xfull_ref.at[my], copy_sem)
        own.start()
        own.wait()

        sends = []
        sidx = [0]

        def send_chunk(slot, tgt, rsem, cond):
            i = sidx[0]
            sidx[0] += 1
            rdma = pltpu.make_async_remote_copy(
                src_ref=xfull_ref.at[slot],
                dst_ref=xfull_ref.at[slot],
                send_sem=send_sems.at[i],
                recv_sem=rsem,
                device_id=(tgt,),
                device_id_type=pl.DeviceIdType.MESH,
            )
            if cond is None:
                rdma.start()
            else:
                @pl.when(cond)
                def _():
                    rdma.start()
            sends.append((cond, rdma))

        def send_inplane(slot, zo, cond):
            send_chunk(slot, x_nbr, ipx_sems.at[zo], cond)
            send_chunk(slot, y_nbr, ipy_sems.at[zo], cond)
            send_chunk(slot, d_nbr, ipd_sems.at[zo], cond)

        def wait_chunk(slot, rsem):
            pltpu.make_async_remote_copy(
                src_ref=src_ref,
                dst_ref=xfull_ref.at[slot],
                send_sem=send_sems.at[0],
                recv_sem=rsem,
                device_id=(my,),
                device_id_type=pl.DeviceIdType.MESH,
            ).wait_recv()

        send_chunk(my, up, up_sems.at[0], has_up)
        send_chunk(my, dn, dn_sems.at[0], has_dn)
        send_inplane(my, z, None)

        for s in range(NZ - 1):
            zo_u = z - 1 - s
            cond_ur = jnp.logical_and(has_dn, zo_u >= 0)
            zo_u = jnp.maximum(zo_u, 0)
            slot_u = NQ * zo_u + q

            @pl.when(cond_ur)
            def _(slot_u=slot_u, s=s):
                wait_chunk(slot_u, up_sems.at[s])
            if s + 1 < NZ - 1:
                send_chunk(slot_u, up, up_sems.at[s + 1],
                           jnp.logical_and(has_up, cond_ur))
            send_inplane(slot_u, zo_u, cond_ur)

            zo_d = z + 1 + s
            cond_dr = jnp.logical_and(has_up, zo_d <= NZ - 1)
            zo_d = jnp.minimum(zo_d, NZ - 1)
            slot_d = NQ * zo_d + q

            @pl.when(cond_dr)
            def _(slot_d=slot_d, s=s):
                wait_chunk(slot_d, dn_sems.at[s])
            if s + 1 < NZ - 1:
                send_chunk(slot_d, dn, dn_sems.at[s + 1],
                           jnp.logical_and(has_dn, cond_dr))
            send_inplane(slot_d, zo_d, cond_dr)

        for zo in range(NZ):
            wait_chunk(NQ * zo + (q ^ 1), ipx_sems.at[zo])
        for zo in range(NZ):
            wait_chunk(NQ * zo + (3 - q), ipy_sems.at[zo])
        for zo in range(NZ):
            wait_chunk(NQ * zo + (q ^ 2), ipd_sems.at[zo])

        xf = xfull_ref[...].reshape(N_DEV * m_per, k)
        wb = w_ref[...].astype(jnp.bfloat16)
        out_ref[...] = jnp.dot(xf, wb, preferred_element_type=jnp.float32)

        for cond, rdma in sends:
            if cond is None:
                rdma.wait_send()
            else:
                @pl.when(cond)
                def _(rdma=rdma):
                    rdma.wait_send()

        assert sidx[0] == N_SEND, sidx[0]

    return pl.pallas_call(
        body,
        out_shape=jax.ShapeDtypeStruct((N_DEV * m_per, n_per), jnp.float32),
        in_specs=[
            pl.BlockSpec(memory_space=pltpu.VMEM),
            pl.BlockSpec(memory_space=pltpu.VMEM),
        ],
        out_specs=pl.BlockSpec(memory_space=pltpu.VMEM),
        scratch_shapes=[
            pltpu.VMEM((N_DEV, m_per, k), jnp.bfloat16),
            pltpu.VMEM((m_per, k), jnp.bfloat16),
            pltpu.SemaphoreType.DMA((N_SEND,)),
            pltpu.SemaphoreType.DMA((NZ - 1,)),
            pltpu.SemaphoreType.DMA((NZ - 1,)),
            pltpu.SemaphoreType.DMA((NZ,)),
            pltpu.SemaphoreType.DMA((NZ,)),
            pltpu.SemaphoreType.DMA((NZ,)),
            pltpu.SemaphoreType.DMA,
        ],
        compiler_params=pltpu.CompilerParams(collective_id=0),
    )(x, w_mat)
